# baseline (speedup 1.0000x reference)
import functools

import jax
import jax.numpy as jnp
from jax import lax
from jax.experimental import pallas as pl
from jax.experimental.pallas import tpu as pltpu
from jax.experimental.pallas import tpu_sc as plsc

_NUM_CORES = 2
_NUM_SUBCORES = 16
_NW = _NUM_CORES * _NUM_SUBCORES
_LANES = 16
_NBUF = 2


@functools.lru_cache(maxsize=None)
def _make_sc_kernel(batch, seq, d):
    ch = seq
    bpw = batch // _NW
    np_ = 8
    ph = bpw // np_
    nb = _NBUF
    wd = 2 * d
    mesh = plsc.VectorSubcoreMesh(
        core_axis_name="c", subcore_axis_name="s",
        num_cores=_NUM_CORES, num_subcores=_NUM_SUBCORES)

    offs = sorted({min(cc * _LANES, ch - _LANES)
                   for cc in range((ch + _LANES - 1) // _LANES)})

    @functools.partial(
        pl.kernel,
        mesh=mesh,
        out_type=jax.ShapeDtypeStruct((batch, seq, d), jnp.float32),
        scratch_types=[
            pltpu.VMEM((2, ph, ch), jnp.int32),
            pltpu.VMEM((256,), jnp.int32),
            pltpu.VMEM((256,), jnp.int32),
            pltpu.VMEM((nb, ch, wd), jnp.float32),
            pltpu.VMEM((ch, d), jnp.float32),
            pltpu.VMEM((ch, d), jnp.float32),
            pltpu.SemaphoreType.DMA,
            pltpu.SemaphoreType.DMA((nb,)),
            pltpu.SemaphoreType.DMA,
        ],
        compiler_params=pltpu.CompilerParams(use_tc_tiling_on_sc=True),
    )
    def k(table2_hbm, idx_hbm, pos_hbm, out_hbm, idx_v, widx0, widx1,
          wide_v, outb_v, pos_v, sx, sg, so):
        widxs = (widx0, widx1)
        wid = lax.axis_index("s") * _NUM_CORES + lax.axis_index("c")
        b0 = wid * bpw
        pltpu.sync_copy(pos_hbm, pos_v)
        pltpu.sync_copy(idx_hbm.at[pl.ds(b0, ph), :], idx_v.at[0])
        pltpu.async_copy(idx_hbm.at[pl.ds(b0 + ph, ph), :], idx_v.at[1], sx)

        def prep_widx(h, jh, bs):
            for off in offs:
                sl = pl.ds(off, _LANES)
                widxs[bs][sl] = lax.shift_right_logical(idx_v[h, jh, sl], 1)

        def gather(bs):
            pltpu.async_copy(
                table2_hbm.at[widxs[bs].at[pl.ds(0, ch)]], wide_v.at[bs],
                sg.at[bs])

        def prep_and_gather(h, jh, b):
            for bs in range(nb):
                @pl.when(b == bs)
                def _(bs=bs):
                    prep_widx(h, jh, bs)
                    gather(bs)

        def wait_gather(b):
            pltpu.make_async_copy(
                table2_hbm.at[widx0.at[pl.ds(0, ch)]], wide_v.at[b],
                sg.at[b]).wait()

        def wait_out():
            pltpu.make_async_copy(outb_v, out_hbm.at[0], so).wait()

        def assemble(h, jh, b):
            prev_end = 0
            for g in range((ch + _LANES - 1) // _LANES):
                off = min(g * _LANES, ch - _LANES)
                parv = lax.rem(idx_v[h, jh, pl.ds(off, _LANES)], 2)
                for t in range(_LANES):
                    s = off + t
                    if s < prev_end:
                        continue
                    odd = parv[t] == 1
                    for cc in range(d // _LANES):
                        sl = pl.ds(cc * _LANES, _LANES)
                        hi = pl.ds(d + cc * _LANES, _LANES)
                        outb_v[s, sl] = (
                            jnp.where(odd, wide_v[b, s, hi], wide_v[b, s, sl])
                            + pos_v[s, sl])
                prev_end = off + _LANES

        prep_widx(0, 0, 0)
        gather(0)

        def chunk_body(j, carry):
            b = j % nb
            part = j // ph
            h = lax.rem(part, 2)
            jh = lax.rem(j, ph)
            wait_gather(b)

            @pl.when(jnp.logical_and(jh == ph - 1, j + 1 < bpw))
            def _():
                pltpu.make_async_copy(
                    idx_hbm.at[pl.ds(0, ph), :], idx_v.at[0], sx).wait()

            @pl.when(j + 1 < bpw)
            def _():
                prep_and_gather(
                    lax.rem((j + 1) // ph, 2), lax.rem(j + 1, ph),
                    (j + 1) % nb)

            @pl.when(j >= 1)
            def _():
                wait_out()

            for bs in range(nb):
                @pl.when(b == bs)
                def _(bs=bs):
                    assemble(h, jh, bs)

            pltpu.async_copy(outb_v, out_hbm.at[b0 + j], so)

            @pl.when(jnp.logical_and(jh == ph - 1, j + ph + 1 < bpw))
            def _():
                pltpu.async_copy(
                    idx_hbm.at[pl.ds(b0 + (part + 2) * ph, ph), :],
                    idx_v.at[h], sx)
            return carry

        lax.fori_loop(0, bpw, chunk_body, 0)
        wait_out()

    return k


def kernel(response_sequence, response_table, positional_table):
    b, s = response_sequence.shape
    v, d = response_table.shape
    table2 = response_table.reshape(v // 2, 2 * d)
    k = _make_sc_kernel(b, s, d)
    return k(table2, response_sequence, positional_table)


# speedup vs baseline: 1.5147x; 1.5147x over previous
import functools

import jax
import jax.numpy as jnp
from jax import lax
from jax.experimental import pallas as pl
from jax.experimental.pallas import tpu as pltpu
from jax.experimental.pallas import tpu_sc as plsc

_NUM_CORES = 2
_NUM_SUBCORES = 16
_NW = _NUM_CORES * _NUM_SUBCORES
_LANES = 16
_NBUF = 4


@functools.lru_cache(maxsize=None)
def _make_sc_kernel(batch, seq, d):
    ch = seq
    bpw = batch // _NW
    nb = _NBUF
    nround = bpw // nb
    assert bpw % nb == 0
    mesh = plsc.VectorSubcoreMesh(
        core_axis_name="c", subcore_axis_name="s",
        num_cores=_NUM_CORES, num_subcores=_NUM_SUBCORES)

    @functools.partial(
        pl.kernel,
        mesh=mesh,
        out_type=jax.ShapeDtypeStruct((batch, seq, d), jnp.float32),
        scratch_types=[
            pltpu.VMEM((bpw, ch), jnp.int32),
            pltpu.VMEM((nb, ch, d), jnp.float32),
            pltpu.VMEM((ch, d), jnp.float32),
            pltpu.SemaphoreType.DMA((nb,)),
            pltpu.SemaphoreType.DMA((nb,)),
        ],
        compiler_params=pltpu.CompilerParams(use_tc_tiling_on_sc=False),
    )
    def k(table_hbm, idx_hbm, pos_hbm, out_hbm, idx_v, rows_v, pos_v, sg, so):
        wid = lax.axis_index("s") * _NUM_CORES + lax.axis_index("c")
        b0 = wid * bpw
        pltpu.sync_copy(idx_hbm.at[pl.ds(b0, bpw), :], idx_v)
        pltpu.sync_copy(pos_hbm, pos_v)

        def gather(j, b):
            pltpu.async_copy(
                table_hbm.at[idx_v.at[j]], rows_v.at[b], sg.at[b])

        def wait_gather(b):
            pltpu.make_async_copy(
                table_hbm.at[idx_v.at[0]], rows_v.at[b], sg.at[b]).wait()

        def wait_out(b):
            pltpu.make_async_copy(
                rows_v.at[b], out_hbm.at[0], so.at[b]).wait()

        for p in range(nb - 1):
            gather(p, p)

        def round_body(g, carry):
            j0 = g * nb
            for b in range(nb):
                j = j0 + b
                wait_gather(b)

                def add_body(r, c2, _b=b):
                    for cc in range(d // _LANES):
                        sl = pl.ds(cc * _LANES, _LANES)
                        plsc.addupdate(rows_v.at[_b, r, sl], pos_v[r, sl])
                    return c2

                lax.fori_loop(0, ch, add_body, 0, unroll=4)
                pltpu.async_copy(rows_v.at[b], out_hbm.at[b0 + j], so.at[b])

                jg = j + nb - 1
                bg = (b - 1) % nb

                @pl.when(jnp.logical_and(jg < bpw, j >= 1))
                def _():
                    wait_out(bg)

                @pl.when(jg < bpw)
                def _():
                    gather(jg, bg)
            return carry

        lax.fori_loop(0, nround, round_body, 0)
        for b in range(nb):
            wait_out(b)

    return k


def kernel(response_sequence, response_table, positional_table):
    b, s = response_sequence.shape
    v, d = response_table.shape
    k = _make_sc_kernel(b, s, d)
    return k(response_table, response_sequence, positional_table)
